# baseline (speedup 1.0000x reference)
import jax
import jax.numpy as jnp
from jax.experimental import pallas as pl
from jax.experimental.pallas import tpu as pltpu

B, C, F, T = 16, 2, 256, 512
G = 4
K = 1024
SUB = 128
BBLK = 8


def _vq_kernel(x_ref, cb_ref, out_ref, out2_ref):
    cb = cb_ref[0]
    cb2 = 0.5 * jnp.sum(cb * cb, axis=1, keepdims=True)
    cb_bf = cb.astype(jnp.bfloat16)
    for i in range(BBLK):
        xs = x_ref[i, 0]
        m = jnp.dot(cb, xs, preferred_element_type=jnp.float32)
        d = cb2 - m
        dmin = jnp.min(d, axis=0, keepdims=True)
        onehot = (d == dmin).astype(jnp.bfloat16)
        q = jax.lax.dot_general(
            cb_bf, onehot, (((0,), (0,)), ((), ())),
            preferred_element_type=jnp.float32)
        out_ref[i, 0] = q
        out2_ref[i, 0] = q


def kernel(x, codebooks):
    xr = x.reshape(B, G, SUB, T)
    out = pl.pallas_call(
        _vq_kernel,
        grid=(G, B // BBLK),
        compiler_params=pltpu.CompilerParams(
            dimension_semantics=("parallel", "parallel")),
        in_specs=[
            pl.BlockSpec((BBLK, 1, SUB, T), lambda g, b: (b, g, 0, 0)),
            pl.BlockSpec((1, K, SUB), lambda g, b: (g, 0, 0)),
        ],
        out_specs=[
            pl.BlockSpec((BBLK, 1, SUB, T), lambda g, b: (b, g, 0, 0)),
            pl.BlockSpec((BBLK, 1, SUB, T), lambda g, b: (b, g, 0, 0)),
        ],
        out_shape=[
            jax.ShapeDtypeStruct((B, G, SUB, T), jnp.float32),
            jax.ShapeDtypeStruct((B, G, SUB, T), jnp.float32),
        ],
    )(xr, codebooks)
    o1, o2 = out
    return (o1.reshape(B, C, F, T), o2.reshape(B, C, F, T))
